# baseline (speedup 1.0000x reference)
import functools

import jax
import jax.numpy as jnp
from jax.experimental import pallas as pl
from jax.experimental.pallas import tpu as pltpu

_TM = 4096
_TC = 128
_LANE = 128
_N_OUT = 3
_OPAD = 8


def _emit_chunks(x_ref, w, b8, o_ref):
    for c in range(_TM // _TC):
        xc = x_ref[c * _TC:(c + 1) * _TC, :].astype(jnp.bfloat16)
        yc = jnp.dot(xc, w, preferred_element_type=jnp.float32)
        s = yc[:, :_OPAD] + b8
        o_ref[:, c * _TC:(c + 1) * _TC] = s.T


def _linear_t2_kernel(xlo_ref, xhi_ref, w_ref, b_ref, olo_ref, ohi_ref):
    w = w_ref[...].astype(jnp.bfloat16)
    b8 = b_ref[0:1, :_OPAD]
    _emit_chunks(xlo_ref, w, b8, olo_ref)
    _emit_chunks(xhi_ref, w, b8, ohi_ref)


@jax.jit
def kernel(representation, w_p, b_p):
    x = representation.astype(jnp.float32)
    B, D = x.shape
    half = B // 2
    n_steps = half // _TM
    grid = (n_steps,)
    yt_lo, yt_hi = pl.pallas_call(
        _linear_t2_kernel,
        out_shape=(
            jax.ShapeDtypeStruct((_OPAD, half), jnp.float32),
            jax.ShapeDtypeStruct((_OPAD, half), jnp.float32),
        ),
        grid=grid,
        in_specs=[
            pl.BlockSpec((_TM, D), lambda i: (i, 0)),
            pl.BlockSpec((_TM, D), lambda i, _n=n_steps: (_n + i, 0)),
            pl.BlockSpec((D, _LANE), lambda i: (0, 0)),
            pl.BlockSpec((1, _LANE), lambda i: (0, 0)),
        ],
        out_specs=(
            pl.BlockSpec((_OPAD, _TM), lambda i: (0, i)),
            pl.BlockSpec((_OPAD, _TM), lambda i: (0, i)),
        ),
        compiler_params=pltpu.CompilerParams(
            dimension_semantics=("parallel",)),
        cost_estimate=pl.CostEstimate(
            flops=2 * B * D * _LANE,
            transcendentals=0,
            bytes_accessed=(B * D + D * _LANE + _LANE + B * _OPAD) * 4,
        ),
    )(x, x, w_p, b_p)
    yt = jnp.concatenate([yt_lo[:_N_OUT, :], yt_hi[:_N_OUT, :]], axis=1)
    return yt.T


# speedup vs baseline: 1.2694x; 1.2694x over previous
import functools

import jax
import jax.numpy as jnp
from jax.experimental import pallas as pl
from jax.experimental.pallas import tpu as pltpu

_TM = 8192
_TC = 128
_LANE = 128
_N_OUT = 3
_OPAD = 8


def _linear_t_kernel(x_ref, w_ref, b_ref, o_ref):
    w = w_ref[...].astype(jnp.bfloat16)
    b8 = b_ref[0:1, :_OPAD]
    for c in range(_TM // _TC):
        xc = x_ref[c * _TC:(c + 1) * _TC, :].astype(jnp.bfloat16)
        yc = jnp.dot(xc, w, preferred_element_type=jnp.float32)
        s = yc[:, :_OPAD] + b8
        st = s.T
        o_ref[:, c * _TC:(c + 1) * _TC] = st[:_N_OUT, :]


@jax.jit
def kernel(representation, w_p, b_p):
    x = representation.astype(jnp.float32)
    B, D = x.shape
    grid = (pl.cdiv(B, _TM),)
    yt = pl.pallas_call(
        _linear_t_kernel,
        out_shape=jax.ShapeDtypeStruct((_N_OUT, B), jnp.float32),
        grid=grid,
        in_specs=[
            pl.BlockSpec((_TM, D), lambda i: (i, 0)),
            pl.BlockSpec((D, _LANE), lambda i: (0, 0)),
            pl.BlockSpec((1, _LANE), lambda i: (0, 0)),
        ],
        out_specs=pl.BlockSpec((_N_OUT, _TM), lambda i: (0, i)),
        compiler_params=pltpu.CompilerParams(
            dimension_semantics=("parallel",)),
        cost_estimate=pl.CostEstimate(
            flops=2 * B * D * _LANE,
            transcendentals=0,
            bytes_accessed=(B * D + D * _LANE + _LANE + B * _N_OUT) * 4,
        ),
    )(x, w_p, b_p)
    return yt.T
